# baseline (speedup 1.0000x reference)
import functools

import jax
import jax.numpy as jnp
from jax import lax
from jax.experimental import pallas as pl
from jax.experimental.pallas import tpu as pltpu
from jax.experimental.pallas import tpu_sc as plsc

N = 10000
NP = 10240
E = 320000
H = 128
NC = 2
NS = 16
NW = NC * NS
KE = 128
EPAD = 327680
EPTP = EPAD // NW
NITE = EPTP // KE
R = 2
NITH = NITE // R
RPT = NP // NS


def _sc_mesh():
    return plsc.VectorSubcoreMesh(core_axis_name="c", subcore_axis_name="s")


@functools.partial(
    pl.kernel,
    out_type=jax.ShapeDtypeStruct((NC * NP, 16), jnp.float32),
    mesh=_sc_mesh(),
    scratch_types=[
        pltpu.VMEM((NITE, KE), jnp.int32),
        pltpu.VMEM((KE, 16), jnp.float32),
        pltpu.VMEM_SHARED((NP, 16), jnp.float32),
        pltpu.SemaphoreType.DMA,
        pltpu.SemaphoreType.DMA,
        pltpu.SemaphoreType.DMA,
        pltpu.SemaphoreType.DMA,
    ],
)
def _deg_kernel(dst_hbm, zeros_hbm, ones_hbm, out_hbm, dst_v, ones_v, acc,
                d0, d1, d2, d3):
    dsem = (d0, d1, d2, d3)
    c = lax.axis_index("c")
    s = lax.axis_index("s")
    wid = c * NS + s
    pltpu.sync_copy(dst_hbm.at[wid], dst_v)
    pltpu.sync_copy(ones_hbm, ones_v)
    pltpu.sync_copy(zeros_hbm.at[pl.ds(s * RPT, RPT)], acc.at[pl.ds(s * RPT, RPT)])
    plsc.subcore_barrier()

    def body(i, carry):
        for r in range(R):
            j = R * i + r

            @pl.when(i > 0)
            def _(r=r, j=j):
                pltpu.make_async_copy(ones_v, acc.at[dst_v.at[j]], dsem[r]).wait()

            pltpu.async_copy(ones_v, acc.at[dst_v.at[j]], dsem[r], add=True)
        return carry

    lax.fori_loop(0, NITH, body, 0)
    for r in range(R):
        pltpu.make_async_copy(ones_v, acc.at[dst_v.at[r]], dsem[r]).wait()
    plsc.subcore_barrier()
    pltpu.sync_copy(acc.at[pl.ds(s * RPT, RPT)],
                    out_hbm.at[pl.ds(c * NP + s * RPT, RPT)])


@functools.partial(
    pl.kernel,
    out_type=jax.ShapeDtypeStruct((NC * NP, H), jnp.float32),
    mesh=_sc_mesh(),
    scratch_types=[
        pltpu.VMEM((KE,), jnp.int32),
        pltpu.VMEM((KE,), jnp.int32),
        pltpu.VMEM((KE,), jnp.int32),
        pltpu.VMEM((KE,), jnp.int32),
        pltpu.VMEM((KE, H), jnp.float32),
        pltpu.VMEM((KE, H), jnp.float32),
        pltpu.VMEM_SHARED((NP, H), jnp.float32),
        pltpu.SemaphoreType.DMA,
        pltpu.SemaphoreType.DMA,
        pltpu.SemaphoreType.DMA,
        pltpu.SemaphoreType.DMA,
    ],
)
def _edge_kernel(hs_hbm, src_hbm, dst_hbm, zeros_hbm, out_hbm,
                 sb0, sb1, db0, db1, rows0, rows1, acc,
                 g0, g1, s0, s1):
    sbuf = (sb0, sb1)
    dbuf = (db0, db1)
    rows = (rows0, rows1)
    gsem = (g0, g1)
    ssem = (s0, s1)
    c = lax.axis_index("c")
    s = lax.axis_index("s")
    wid = c * NS + s
    base0 = wid * EPTP
    pltpu.sync_copy(zeros_hbm.at[pl.ds(s * RPT, RPT)], acc.at[pl.ds(s * RPT, RPT)])
    for r in range(R):
        pltpu.sync_copy(src_hbm.at[pl.ds(base0 + r * KE, KE)], sbuf[r])
        pltpu.sync_copy(dst_hbm.at[pl.ds(base0 + r * KE, KE)], dbuf[r])
        pltpu.async_copy(hs_hbm.at[sbuf[r]], rows[r], gsem[r])
    plsc.subcore_barrier()

    def body(i, carry):
        scat = []
        for r in range(R):
            pltpu.make_async_copy(hs_hbm.at[sbuf[r]], rows[r], gsem[r]).wait()
            scat.append(
                pltpu.async_copy(rows[r], acc.at[dbuf[r]], ssem[r], add=True))
        for r in range(R):
            nxt = base0 + (R * i + R + r) * KE

            @pl.when(i < NITH - 1)
            def _(r=r, nxt=nxt):
                pltpu.sync_copy(src_hbm.at[pl.ds(nxt, KE)], sbuf[r])
            scat[r].wait()

            @pl.when(i < NITH - 1)
            def _(r=r, nxt=nxt):
                pltpu.sync_copy(dst_hbm.at[pl.ds(nxt, KE)], dbuf[r])
                pltpu.async_copy(hs_hbm.at[sbuf[r]], rows[r], gsem[r])
        return carry

    lax.fori_loop(0, NITH, body, 0)
    plsc.subcore_barrier()
    pltpu.sync_copy(acc.at[pl.ds(s * RPT, RPT)],
                    out_hbm.at[pl.ds(c * NP + s * RPT, RPT)])


B = 1000


def _mm1_body(degp_ref, x_ref, w_ref, hs_ref, dinv_ref):
    deg = degp_ref[0, :, 0:1] + 1.0
    for _c in range(1, NC):
        deg = deg + degp_ref[_c, :, 0:1]
    dinv = lax.rsqrt(deg)
    h = jnp.dot(x_ref[...], w_ref[...], preferred_element_type=jnp.float32)
    hs_ref[...] = dinv * h
    dinv_ref[...] = dinv


_mm1 = pl.pallas_call(
    _mm1_body,
    grid=(N // B,),
    in_specs=[
        pl.BlockSpec((NC, B, 16), lambda i: (0, i, 0)),
        pl.BlockSpec((B, H), lambda i: (i, 0)),
        pl.BlockSpec((H, H), lambda i: (0, 0)),
    ],
    out_specs=[
        pl.BlockSpec((B, H), lambda i: (i, 0)),
        pl.BlockSpec((B, 1), lambda i: (i, 0)),
    ],
    out_shape=[
        jax.ShapeDtypeStruct((N, H), jnp.float32),
        jax.ShapeDtypeStruct((N, 1), jnp.float32),
    ],
)


def _lay_body(sp_ref, hs_ref, dinv_ref, b_ref, g_ref, w_ref, out_ref):
    dinv = dinv_ref[...]
    g = g_ref[...]
    t = sp_ref[0] + hs_ref[...]
    for _c in range(1, NC):
        t = t + sp_ref[_c]
    z = dinv * t + b_ref[...]
    a = g * jnp.maximum(z, 0.0) + (1.0 - g) * z
    fac = g * dinv + (1.0 - g)
    out_ref[...] = fac * jnp.dot(a, w_ref[...], preferred_element_type=jnp.float32)


_mm_lay = pl.pallas_call(
    _lay_body,
    grid=(N // B,),
    in_specs=[
        pl.BlockSpec((NC, B, H), lambda i: (0, i, 0)),
        pl.BlockSpec((B, H), lambda i: (i, 0)),
        pl.BlockSpec((B, 1), lambda i: (i, 0)),
        pl.BlockSpec((1, H), lambda i: (0, 0)),
        pl.BlockSpec((1, H), lambda i: (0, 0)),
        pl.BlockSpec((H, H), lambda i: (0, 0)),
    ],
    out_specs=pl.BlockSpec((B, H), lambda i: (i, 0)),
    out_shape=jax.ShapeDtypeStruct((N, H), jnp.float32),
)


def kernel(edge_index, emb, W1, b1, W2, b2, W3, b3):
    src = edge_index[0]
    dst = edge_index[1]
    pad = EPAD - E
    srcp = jnp.concatenate([src, jnp.zeros((pad,), src.dtype)])
    dstp = jnp.concatenate([dst, jnp.full((pad,), N, dst.dtype)])
    dstp3 = dstp.reshape(NW, NITE, KE)
    zeros16 = jnp.zeros((NP, 16), jnp.float32)
    zerosH = jnp.zeros((NP, H), jnp.float32)
    ones = jnp.ones((KE, 16), jnp.float32)

    degp = _deg_kernel(dstp3, zeros16, ones).reshape(NC, NP, 16)[:, :N, :]
    hs1, dinv = _mm1(degp, emb, W1)

    one_g = jnp.ones((1, H), jnp.float32)
    zero_g = jnp.zeros((1, H), jnp.float32)
    eye = jnp.eye(H, dtype=jnp.float32)

    s1 = _edge_kernel(hs1, srcp, dstp, zerosH).reshape(NC, NP, H)[:, :N, :]
    hs2 = _mm_lay(s1, hs1, dinv, b1.reshape(1, H), one_g, W2)
    s2 = _edge_kernel(hs2, srcp, dstp, zerosH).reshape(NC, NP, H)[:, :N, :]
    hs3 = _mm_lay(s2, hs2, dinv, b2.reshape(1, H), one_g, W3)
    s3 = _edge_kernel(hs3, srcp, dstp, zerosH).reshape(NC, NP, H)[:, :N, :]
    out = _mm_lay(s3, hs3, dinv, b3.reshape(1, H), zero_g, eye)
    return out
